# baseline (device time: 11780 ns/iter reference)
import jax
import jax.numpy as jnp
from jax import lax
from jax.experimental import pallas as pl
from jax.experimental.pallas import tpu as pltpu

K = 5
CH = 32
F = K * CH
D0 = 2 * F
_MESH = pl.DeviceIdType.MESH


def kernel(x):
    m, n = x.shape
    half = n // 2
    xb = x.astype(jnp.bfloat16)

    def body(x_ref, out_ref, blk, loc_sems, a_send, a_recv, b_send, b_recv):
        my_x = lax.axis_index("x")
        my_y = lax.axis_index("y")
        peer_y = 1 - my_y
        peer_x = 1 - my_x

        barrier = pltpu.get_barrier_semaphore()
        pl.semaphore_signal(barrier, inc=1, device_id=(my_x, peer_y),
                            device_id_type=_MESH)
        pl.semaphore_signal(barrier, inc=1, device_id=(peer_x, my_y),
                            device_id_type=_MESH)
        pl.semaphore_wait(barrier, 2)

        send_col = peer_y * half
        ydst = my_y * m
        my_base = peer_y * m
        foff = my_x * F

        cp_corner = pltpu.make_async_copy(
            x_ref.at[:, pl.ds(my_y * half, half)],
            out_ref.at[pl.ds(my_y * m, m), :],
            loc_sems.at[K],
        )
        cp_corner.start()

        a_rdmas = []
        for k in range(K):
            r = pltpu.make_async_remote_copy(
                src_ref=x_ref.at[pl.ds(foff + k * CH, CH), pl.ds(send_col, half)],
                dst_ref=blk.at[pl.ds(foff + k * CH, CH), :],
                send_sem=a_send.at[k],
                recv_sem=a_recv.at[k],
                device_id=(my_x, peer_y),
                device_id_type=_MESH,
            )
            r.start()
            a_rdmas.append(r)
        d_rdma = pltpu.make_async_remote_copy(
            src_ref=x_ref.at[pl.ds(D0, m - D0), pl.ds(send_col, half)],
            dst_ref=out_ref.at[pl.ds(ydst + D0, m - D0), :],
            send_sem=a_send.at[K],
            recv_sem=a_recv.at[K],
            device_id=(my_x, peer_y),
            device_id_type=_MESH,
        )
        d_rdma.start()

        f_rdmas = []
        for k in range(K):
            a_rdmas[k].wait_recv()
            r = pltpu.make_async_remote_copy(
                src_ref=blk.at[pl.ds(foff + k * CH, CH), :],
                dst_ref=out_ref.at[pl.ds(my_base + foff + k * CH, CH), :],
                send_sem=b_send.at[k],
                recv_sem=b_recv.at[k],
                device_id=(peer_x, my_y),
                device_id_type=_MESH,
            )
            r.start()
            f_rdmas.append(r)
            pltpu.make_async_copy(
                blk.at[pl.ds(foff + k * CH, CH), :],
                out_ref.at[pl.ds(my_base + foff + k * CH, CH), :],
                loc_sems.at[k],
            ).start()

        d_rdma.wait_recv()
        for k in range(K):
            f_rdmas[k].wait_recv()
        cp_corner.wait()
        for k in range(K):
            pltpu.make_async_copy(
                blk.at[pl.ds(foff + k * CH, CH), :],
                out_ref.at[pl.ds(my_base + foff + k * CH, CH), :],
                loc_sems.at[k],
            ).wait()
        for k in range(K):
            a_rdmas[k].wait_send()
            f_rdmas[k].wait_send()
        d_rdma.wait_send()

    out_shape = jax.ShapeDtypeStruct((2 * m, half), jnp.bfloat16)
    return pl.pallas_call(
        body,
        out_shape=out_shape,
        in_specs=[pl.BlockSpec(memory_space=pltpu.VMEM)],
        out_specs=pl.BlockSpec(memory_space=pltpu.MemorySpace.HBM),
        scratch_shapes=[
            pltpu.VMEM((2 * F, half), jnp.bfloat16),
            pltpu.SemaphoreType.DMA((K + 1,)),
            pltpu.SemaphoreType.DMA((K + 1,)),
            pltpu.SemaphoreType.DMA((K + 1,)),
            pltpu.SemaphoreType.DMA((K,)),
            pltpu.SemaphoreType.DMA((K,)),
        ],
        compiler_params=pltpu.CompilerParams(collective_id=0),
    )(xb)


# device time: 11634 ns/iter; 1.0125x vs baseline; 1.0125x over previous
import jax
import jax.numpy as jnp
from jax import lax
from jax.experimental import pallas as pl
from jax.experimental.pallas import tpu as pltpu

K = 5
CH = 32
F = K * CH
D0 = 2 * F
_MESH = pl.DeviceIdType.MESH


def kernel(x):
    m, n = x.shape
    half = n // 2
    xb = x.astype(jnp.bfloat16)

    def body(x_ref, out_ref, a_send, a_recv, b_send, b_recv):
        my_x = lax.axis_index("x")
        my_y = lax.axis_index("y")
        peer_y = 1 - my_y
        peer_x = 1 - my_x

        barrier = pltpu.get_barrier_semaphore()
        pl.semaphore_signal(barrier, inc=1, device_id=(my_x, peer_y),
                            device_id_type=_MESH)
        pl.semaphore_signal(barrier, inc=1, device_id=(peer_x, my_y),
                            device_id_type=_MESH)
        pl.semaphore_wait(barrier, 2)

        send_col = peer_y * half
        ydst = my_y * m
        my_base = peer_y * m
        foff = my_x * F

        a_rdmas = []
        for k in range(K):
            r = pltpu.make_async_remote_copy(
                src_ref=x_ref.at[pl.ds(foff + k * CH, CH), pl.ds(send_col, half)],
                dst_ref=out_ref.at[pl.ds(ydst + foff + k * CH, CH), :],
                send_sem=a_send.at[k],
                recv_sem=a_recv.at[k],
                device_id=(my_x, peer_y),
                device_id_type=_MESH,
            )
            r.start()
            a_rdmas.append(r)
        d_rdma = pltpu.make_async_remote_copy(
            src_ref=x_ref.at[pl.ds(D0, m - D0), pl.ds(send_col, half)],
            dst_ref=out_ref.at[pl.ds(ydst + D0, m - D0), :],
            send_sem=a_send.at[K],
            recv_sem=a_recv.at[K],
            device_id=(my_x, peer_y),
            device_id_type=_MESH,
        )
        d_rdma.start()

        out_ref[pl.ds(my_y * m, m), :] = x_ref[:, pl.ds(my_y * half, half)]

        f_rdmas = []
        for k in range(K):
            a_rdmas[k].wait_recv()
            r = pltpu.make_async_remote_copy(
                src_ref=out_ref.at[pl.ds(my_base + foff + k * CH, CH), :],
                dst_ref=out_ref.at[pl.ds(my_base + foff + k * CH, CH), :],
                send_sem=b_send.at[k],
                recv_sem=b_recv.at[k],
                device_id=(peer_x, my_y),
                device_id_type=_MESH,
            )
            r.start()
            f_rdmas.append(r)

        d_rdma.wait_recv()
        for k in range(K):
            f_rdmas[k].wait_recv()
        for k in range(K):
            a_rdmas[k].wait_send()
            f_rdmas[k].wait_send()
        d_rdma.wait_send()

    out_shape = jax.ShapeDtypeStruct((2 * m, half), jnp.bfloat16)
    return pl.pallas_call(
        body,
        out_shape=out_shape,
        in_specs=[pl.BlockSpec(memory_space=pltpu.VMEM)],
        out_specs=pl.BlockSpec(memory_space=pltpu.VMEM),
        scratch_shapes=[
            pltpu.SemaphoreType.DMA((K + 1,)),
            pltpu.SemaphoreType.DMA((K + 1,)),
            pltpu.SemaphoreType.DMA((K,)),
            pltpu.SemaphoreType.DMA((K,)),
        ],
        compiler_params=pltpu.CompilerParams(collective_id=0),
    )(xb)
